# baseline (speedup 1.0000x reference)
import functools
import math

import jax
import jax.numpy as jnp
from jax import lax
from jax.experimental import pallas as pl
from jax.experimental.pallas import tpu as pltpu
from jax.experimental.pallas import tpu_sc as plsc

D_MODEL = 64
SCALE = math.sqrt(D_MODEL)

NC = 2
NS = 16
NW = NC * NS
C = 128
LANES = 16
SUB = 8
ROW = 2 * D_MODEL
NBUF = 4
ABUF = 7
PER_W = 245


def _detile_body(embt_hbm, last_hbm, tbl_hbm, slab, rowb, gsems, ssems):
    wid = lax.axis_index("s") * NC + lax.axis_index("c")
    full_slabs = embt_hbm.shape[1] // C
    n_steps = PER_W // ABUF
    base_k = wid * PER_W

    iota = lax.iota(jnp.int32, LANES)
    lanevecs = [iota + 16 * o for o in range(SUB)]
    rvecs = [(iota >> 1) + 8 * o for o in range(SUB)]
    parvec = (iota & 1) * D_MODEL

    def jv_of(k):
        return jnp.minimum(base_k + k, full_slabs - 1)

    def start_gather(b, k):
        jv = jv_of(k)
        pltpu.async_copy(
            embt_hbm.at[:, pl.ds(jv * C, C)], slab.at[b], gsems.at[b])

    def wait_gather(b, k):
        jv = jv_of(k)
        pltpu.make_async_copy(
            embt_hbm.at[:, pl.ds(jv * C, C)], slab.at[b],
            gsems.at[b]).wait()

    def start_scatter(b, k):
        jv = jv_of(k)
        pltpu.async_copy(rowb.at[b], tbl_hbm.at[pl.ds(jv * D_MODEL, D_MODEL)],
                         ssems.at[b])

    def wait_scatter(b, k):
        jv = jv_of(k)
        pltpu.make_async_copy(
            rowb.at[b], tbl_hbm.at[pl.ds(jv * D_MODEL, D_MODEL)],
            ssems.at[b]).wait()

    def transpose(b, rows):
        src = slab.at[b]
        dst = rowb.at[b]

        @plsc.parallel_loop(0, D_MODEL, unroll=8)
        def per_d(d0):
            dvec = (d0 + iota) & (D_MODEL - 1)
            cvec = parvec + dvec
            for o in range(rows // 8):
                vals = plsc.load_gather(src, [dvec, lanevecs[o]])
                plsc.store_scatter(dst, [rvecs[o], cvec], vals)

    for b in range(ABUF):
        start_gather(b, b)

    def step(s, carry):
        k0 = s * ABUF
        for b in range(ABUF):
            k = k0 + b
            wait_gather(b, k)

            @pl.when(s > 0)
            def _():
                wait_scatter(b, k - ABUF)

            transpose(b, D_MODEL)

            @pl.when(s < n_steps - 1)
            def _():
                start_gather(b, k + ABUF)

            start_scatter(b, k)
        return carry

    lax.fori_loop(0, n_steps, step, 0)
    for b in range(ABUF):
        wait_scatter(b, (n_steps - 1) * ABUF + b)

    @pl.when(wid == NW - 1)
    def _():
        pltpu.sync_copy(last_hbm, slab.at[0])
        transpose(0, D_MODEL // 2)
        pltpu.sync_copy(rowb.at[0],
                        tbl_hbm.at[pl.ds(full_slabs * D_MODEL, D_MODEL)])


def _lookup_body(xt_hbm, tbl_hbm, out_hbm, idx_v, idx_h, rin, tbuf, gsems,
                 ssems):
    wid = lax.axis_index("s") * NC + lax.axis_index("c")
    g_tot = xt_hbm.shape[1]
    n_steps = g_tot // NBUF
    jtiles = out_hbm.shape[2]

    pltpu.sync_copy(xt_hbm.at[wid], idx_v)

    @plsc.parallel_loop(0, g_tot, unroll=2)
    def halve(g):
        for jj in range(C // LANES):
            sl = pl.ds(jj * LANES, LANES)
            idx_h[g, sl] = lax.shift_right_logical(idx_v[g, sl], 1)

    iota = lax.iota(jnp.int32, LANES)
    toks = [iota + (jj * LANES) for jj in range(C // LANES)]

    def start_gather(b, g):
        pltpu.async_copy(tbl_hbm.at[idx_h.at[g]], rin.at[b], gsems.at[b])

    def wait_gather(b, g):
        pltpu.make_async_copy(tbl_hbm.at[idx_h.at[g]], rin.at[b],
                              gsems.at[b]).wait()

    def start_scatter(b, g):
        c = wid * g_tot + g
        h = c // jtiles
        j = lax.rem(c, jtiles)
        for i in range(D_MODEL // SUB):
            pltpu.async_copy(tbuf.at[b, pl.ds(i * SUB, SUB)],
                             out_hbm.at[h, i, j], ssems.at[b])

    def wait_scatter(b, g):
        c = wid * g_tot + g
        h = c // jtiles
        j = lax.rem(c, jtiles)
        for i in range(D_MODEL // SUB):
            pltpu.make_async_copy(tbuf.at[b, pl.ds(i * SUB, SUB)],
                                  out_hbm.at[h, i, j], ssems.at[b]).wait()

    def transpose_scale(b, g):
        src = rin.at[b]
        bases = []
        for jj in range(C // LANES):
            iv = idx_v[g, pl.ds(jj * LANES, LANES)]
            bases.append(lax.shift_left((iv & 1), 6))

        dst = tbuf.at[b]

        @plsc.parallel_loop(0, D_MODEL, unroll=8)
        def per_d(d0):
            dvec = (d0 + iota) & (D_MODEL - 1)
            for jj in range(C // LANES):
                vals = plsc.load_gather(src, [toks[jj], bases[jj] + dvec])
                plsc.store_scatter(dst, [dvec, toks[jj]], vals * SCALE)

    for b in range(NBUF):
        start_gather(b, b)

    def step(s, carry):
        g0 = s * NBUF
        for b in range(NBUF):
            g = g0 + b
            wait_gather(b, g)

            @pl.when(s > 0)
            def _():
                wait_scatter(b, g - NBUF)

            transpose_scale(b, g)

            @pl.when(s < n_steps - 1)
            def _():
                start_gather(b, g + NBUF)

            start_scatter(b, g)
        return carry

    lax.fori_loop(0, n_steps, step, 0)

    g0 = (n_steps - 1) * NBUF
    for b in range(NBUF):
        wait_scatter(b, g0 + b)


def kernel(x, embedding):
    bsz, h = x.shape
    n = bsz * h
    assert n % (NW * C * NBUF) == 0
    g_per_w = n // (NW * C)
    jtiles = bsz // C
    vocab = embedding.shape[0]

    mesh = plsc.VectorSubcoreMesh(core_axis_name="c", subcore_axis_name="s")

    embt = embedding.T
    tail = vocab % C
    nrows = (vocab // C) * D_MODEL + D_MODEL
    last = jnp.pad(embt[:, vocab - tail:], ((0, 0), (0, C - tail)))
    tbl = pl.kernel(
        _detile_body,
        out_type=jax.ShapeDtypeStruct((nrows, C), jnp.float32),
        mesh=mesh,
        scratch_types=[
            pltpu.VMEM((ABUF, D_MODEL, C), jnp.float32),
            pltpu.VMEM((ABUF, D_MODEL, C), jnp.float32),
            pltpu.SemaphoreType.DMA((ABUF,)),
            pltpu.SemaphoreType.DMA((ABUF,)),
        ],
        compiler_params=pltpu.CompilerParams(
            use_tc_tiling_on_sc=True, needs_layout_passes=False),
    )(embt, last)

    xt = x.T.astype(jnp.int32).reshape(NW, g_per_w, C)
    out5 = pl.kernel(
        _lookup_body,
        out_type=jax.ShapeDtypeStruct(
            (h, D_MODEL // SUB, jtiles, SUB, C), jnp.float32),
        mesh=mesh,
        scratch_types=[
            pltpu.VMEM((g_per_w, C), jnp.int32),
            pltpu.VMEM((g_per_w, C), jnp.int32),
            pltpu.VMEM((NBUF, C, ROW), jnp.float32),
            pltpu.VMEM((NBUF, D_MODEL, C), jnp.float32),
            pltpu.SemaphoreType.DMA((NBUF,)),
            pltpu.SemaphoreType.DMA((NBUF,)),
        ],
        compiler_params=pltpu.CompilerParams(
            use_tc_tiling_on_sc=False, needs_layout_passes=False),
    )(xt, tbl)
    return out5.transpose(2, 4, 0, 1, 3).reshape(bsz, h, D_MODEL)


# speedup vs baseline: 1.0069x; 1.0069x over previous
import functools
import math

import jax
import jax.numpy as jnp
from jax import lax
from jax.experimental import pallas as pl
from jax.experimental.pallas import tpu as pltpu
from jax.experimental.pallas import tpu_sc as plsc

D_MODEL = 64
SCALE = math.sqrt(D_MODEL)

NC = 2
NS = 16
NW = NC * NS
C = 128
LANES = 16
SUB = 8
ROW = 2 * D_MODEL
NBUF = 4
ABUF = 5
PER_W = 245


def _detile_body(embt_hbm, last_hbm, tbl_hbm, slab, rowb, gsems, ssems):
    wid = lax.axis_index("s") * NC + lax.axis_index("c")
    full_slabs = embt_hbm.shape[1] // C
    n_steps = PER_W // ABUF
    base_k = wid * PER_W

    iota = lax.iota(jnp.int32, LANES)
    lanevecs = [iota + 16 * o for o in range(SUB)]
    rvecs = [(iota >> 1) + 8 * o for o in range(SUB)]
    parvec = (iota & 1) * D_MODEL

    def jv_of(k):
        return jnp.minimum(base_k + k, full_slabs - 1)

    def start_gather(b, k):
        jv = jv_of(k)
        pltpu.async_copy(
            embt_hbm.at[:, pl.ds(jv * C, C)], slab.at[b], gsems.at[b])

    def wait_gather(b, k):
        jv = jv_of(k)
        pltpu.make_async_copy(
            embt_hbm.at[:, pl.ds(jv * C, C)], slab.at[b],
            gsems.at[b]).wait()

    def start_scatter(b, k):
        jv = jv_of(k)
        pltpu.async_copy(rowb.at[b], tbl_hbm.at[pl.ds(jv * D_MODEL, D_MODEL)],
                         ssems.at[b])

    def wait_scatter(b, k):
        jv = jv_of(k)
        pltpu.make_async_copy(
            rowb.at[b], tbl_hbm.at[pl.ds(jv * D_MODEL, D_MODEL)],
            ssems.at[b]).wait()

    def transpose(b, rows):
        src = slab.at[b]
        dst = rowb.at[b]

        @plsc.parallel_loop(0, D_MODEL, unroll=8)
        def per_d(d0):
            dvec = (d0 + iota) & (D_MODEL - 1)
            cvec = parvec + dvec
            for o in range(rows // 8):
                vals = plsc.load_gather(src, [dvec, lanevecs[o]])
                plsc.store_scatter(dst, [rvecs[o], cvec], vals)

    for b in range(ABUF):
        start_gather(b, b)

    def step(s, carry):
        k0 = s * ABUF
        for b in range(ABUF):
            k = k0 + b
            wait_gather(b, k)

            @pl.when(s > 0)
            def _():
                wait_scatter(b, k - ABUF)

            transpose(b, D_MODEL)

            @pl.when(s < n_steps - 1)
            def _():
                start_gather(b, k + ABUF)

            start_scatter(b, k)
        return carry

    lax.fori_loop(0, n_steps, step, 0)
    for b in range(ABUF):
        wait_scatter(b, (n_steps - 1) * ABUF + b)

    @pl.when(wid == NW - 1)
    def _():
        pltpu.sync_copy(last_hbm, slab.at[0])
        transpose(0, D_MODEL // 2)
        pltpu.sync_copy(rowb.at[0],
                        tbl_hbm.at[pl.ds(full_slabs * D_MODEL, D_MODEL)])


def _lookup_body(xt_hbm, tbl_hbm, out_hbm, idx_v, idx_h, rin, tbuf, gsems,
                 ssems):
    wid = lax.axis_index("s") * NC + lax.axis_index("c")
    g_tot = xt_hbm.shape[1]
    n_steps = g_tot // NBUF
    jtiles = out_hbm.shape[2]

    pltpu.sync_copy(xt_hbm.at[wid], idx_v)

    @plsc.parallel_loop(0, g_tot, unroll=2)
    def halve(g):
        for jj in range(C // LANES):
            sl = pl.ds(jj * LANES, LANES)
            idx_h[g, sl] = lax.shift_right_logical(idx_v[g, sl], 1)

    iota = lax.iota(jnp.int32, LANES)
    toks = [iota + (jj * LANES) for jj in range(C // LANES)]

    def start_gather(b, g):
        pltpu.async_copy(tbl_hbm.at[idx_h.at[g]], rin.at[b], gsems.at[b])

    def wait_gather(b, g):
        pltpu.make_async_copy(tbl_hbm.at[idx_h.at[g]], rin.at[b],
                              gsems.at[b]).wait()

    def start_scatter(b, g):
        c = wid * g_tot + g
        h = c // jtiles
        j = lax.rem(c, jtiles)
        for i in range(D_MODEL // SUB):
            pltpu.async_copy(tbuf.at[b, pl.ds(i * SUB, SUB)],
                             out_hbm.at[h, i, j], ssems.at[b])

    def wait_scatter(b, g):
        c = wid * g_tot + g
        h = c // jtiles
        j = lax.rem(c, jtiles)
        for i in range(D_MODEL // SUB):
            pltpu.make_async_copy(tbuf.at[b, pl.ds(i * SUB, SUB)],
                                  out_hbm.at[h, i, j], ssems.at[b]).wait()

    def transpose_scale(b, g):
        src = rin.at[b]
        bases = []
        for jj in range(C // LANES):
            iv = idx_v[g, pl.ds(jj * LANES, LANES)]
            bases.append(lax.shift_left((iv & 1), 6))

        dst = tbuf.at[b]

        @plsc.parallel_loop(0, D_MODEL, unroll=8)
        def per_d(d0):
            dvec = (d0 + iota) & (D_MODEL - 1)
            for jj in range(C // LANES):
                vals = plsc.load_gather(src, [toks[jj], bases[jj] + dvec])
                plsc.store_scatter(dst, [dvec, toks[jj]], vals * SCALE)

    for b in range(NBUF):
        start_gather(b, b)

    def step(s, carry):
        g0 = s * NBUF
        for b in range(NBUF):
            g = g0 + b
            wait_gather(b, g)

            @pl.when(s > 0)
            def _():
                wait_scatter(b, g - NBUF)

            transpose_scale(b, g)

            @pl.when(s < n_steps - 1)
            def _():
                start_gather(b, g + NBUF)

            start_scatter(b, g)
        return carry

    lax.fori_loop(0, n_steps, step, 0)

    g0 = (n_steps - 1) * NBUF
    for b in range(NBUF):
        wait_scatter(b, g0 + b)


def kernel(x, embedding):
    bsz, h = x.shape
    n = bsz * h
    assert n % (NW * C * NBUF) == 0
    g_per_w = n // (NW * C)
    jtiles = bsz // C
    vocab = embedding.shape[0]

    mesh = plsc.VectorSubcoreMesh(core_axis_name="c", subcore_axis_name="s")

    embt = embedding.T
    tail = vocab % C
    nrows = (vocab // C) * D_MODEL + D_MODEL
    last = jnp.pad(embt[:, vocab - tail:], ((0, 0), (0, C - tail)))
    tbl = pl.kernel(
        _detile_body,
        out_type=jax.ShapeDtypeStruct((nrows, C), jnp.float32),
        mesh=mesh,
        scratch_types=[
            pltpu.VMEM((ABUF, D_MODEL, C), jnp.float32),
            pltpu.VMEM((ABUF, D_MODEL, C), jnp.float32),
            pltpu.SemaphoreType.DMA((ABUF,)),
            pltpu.SemaphoreType.DMA((ABUF,)),
        ],
        compiler_params=pltpu.CompilerParams(
            use_tc_tiling_on_sc=True, needs_layout_passes=False),
    )(embt, last)

    xt = x.T.astype(jnp.int32).reshape(NW, g_per_w, C)
    out5 = pl.kernel(
        _lookup_body,
        out_type=jax.ShapeDtypeStruct(
            (h, D_MODEL // SUB, jtiles, SUB, C), jnp.float32),
        mesh=mesh,
        scratch_types=[
            pltpu.VMEM((g_per_w, C), jnp.int32),
            pltpu.VMEM((g_per_w, C), jnp.int32),
            pltpu.VMEM((NBUF, C, ROW), jnp.float32),
            pltpu.VMEM((NBUF, D_MODEL, C), jnp.float32),
            pltpu.SemaphoreType.DMA((NBUF,)),
            pltpu.SemaphoreType.DMA((NBUF,)),
        ],
        compiler_params=pltpu.CompilerParams(
            use_tc_tiling_on_sc=False, needs_layout_passes=False),
    )(xt, tbl)
    return out5.transpose(2, 4, 0, 1, 3).reshape(bsz, h, D_MODEL)
